# baseline (speedup 1.0000x reference)
import functools

import jax
import jax.numpy as jnp
from jax import lax
from jax.experimental import pallas as pl
from jax.experimental.pallas import tpu as pltpu
from jax.experimental.pallas import tpu_sc as plsc

N = 2048
D = 768
E = 8
TOPK = 4
HID = 768
SHID = 2 * 768

TILE = 256
GT = 256
NPAD = N * TOPK + E * GT
NT_G = NPAD // GT
NT_G_PAD = ((NT_G + 15) // 16) * 16

NC = 2
NS = 16
NW = NC * NS
TPW = N // NW
CPW = TPW // 16
NCHUNK = N // 16


def _silu(v):
    return v * jax.nn.sigmoid(v)


def _pre_body(x_ref, W0_ref, b0_ref, Wsg0_ref, bsg0_ref, Wg_ref,
              h_ref, comb_ref):
    x = x_ref[...]
    t0 = jnp.dot(x, W0_ref[...], preferred_element_type=jnp.float32) + b0_ref[...]
    z = jnp.dot(t0, Wsg0_ref[...], preferred_element_type=jnp.float32) + bsg0_ref[...]
    h = z[:, :D] * _silu(z[:, D:])
    h_ref[...] = h

    logits = jnp.dot(h, Wg_ref[...], preferred_element_type=jnp.float32)
    m = jnp.max(logits, axis=-1, keepdims=True)
    p = jnp.exp(logits - m)
    s = p / jnp.sum(p, axis=-1, keepdims=True)
    col = lax.broadcasted_iota(jnp.int32, (TILE, E), 1)
    rank = jnp.zeros((TILE, E), jnp.int32)
    for j in range(E):
        sj = s[:, j:j + 1]
        rank = rank + (sj > s).astype(jnp.int32) \
                    + ((sj == s) & (j < col)).astype(jnp.int32)
    comb_ref[...] = jnp.where(rank < TOPK, s, 0.0)


def _shared_body(h_ref, Ws1_ref, Ws3_ref, Ws2_ref, S_ref):
    h = h_ref[...]
    z1 = jnp.dot(h, Ws1_ref[...], preferred_element_type=jnp.float32)
    z3 = jnp.dot(h, Ws3_ref[...], preferred_element_type=jnp.float32)
    S_ref[...] = jnp.dot(_silu(z1) * z3, Ws2_ref[...],
                         preferred_element_type=jnp.float32)


def _sc_dispatch_body(comb_hbm, h_hbm,
                      hs_hbm, posT_hbm, wflat_hbm, texp_hbm,
                      comb_v, pos_v, wflat_v, texp_v, h_v, sem):
    wid = lax.axis_index("s") * NC + lax.axis_index("c")
    base = wid * TPW
    lanes = lax.iota(jnp.int32, 16)
    zeros_i = jnp.zeros((16,), jnp.int32)

    pltpu.sync_copy(comb_hbm, comb_v)

    my_c0 = wid * CPW

    def scan_body(c, carry):
        vsum, vpre = carry
        tok = c * 16 + lanes
        on = (c < my_c0).astype(jnp.int32)
        new_sum = []
        new_pre = []
        for e in range(E):
            v = plsc.load_gather(comb_v, [tok * E + e])
            sel = (v > 0.0).astype(jnp.int32)
            new_sum.append(vsum[e] + sel)
            new_pre.append(vpre[e] + sel * on)
        return tuple(new_sum), tuple(new_pre)

    vsum0 = tuple(zeros_i for _ in range(E))
    vpre0 = tuple(zeros_i for _ in range(E))
    vsum, vpre = lax.fori_loop(0, NCHUNK, scan_body, (vsum0, vpre0))

    totals = [jnp.sum(vsum[e]) for e in range(E)]
    prefix = [jnp.sum(vpre[e]) for e in range(E)]

    off = []
    acc = jnp.int32(0)
    for e in range(E):
        off.append(acc)
        padded = ((totals[e] + (GT - 1)) // GT) * GT
        acc = acc + padded
    my_base = [off[e] + prefix[e] for e in range(E)]

    @pl.when(wid == 0)
    def _write_texp():
        for cidx in range(NT_G_PAD // 16):
            tvec = lanes + 16 * cidx
            cnt = jnp.zeros((16,), jnp.int32)
            for e in range(E):
                cnt = cnt + (tvec >= (off[e] // GT)).astype(jnp.int32)
            texp_v[pl.ds(cidx * 16, 16)] = jnp.minimum(cnt - 1, E - 1)
        pltpu.sync_copy(texp_v, texp_hbm)

    for k in range(TOPK):
        for cc in range(CPW):
            pos_v[k, pl.ds(cc * 16, 16)] = jnp.full((16,), NPAD - 1, jnp.int32)
    for cc in range(CPW * TOPK // 16):
        wflat_v[pl.ds(cc * 16, 16)] = jnp.zeros((16,), jnp.float32)

    cur = list(my_base)
    for c in range(CPW):
        tok_local = c * 16 + lanes
        tok = base + tok_local
        kcount = zeros_i
        for e in range(E):
            v = plsc.load_gather(comb_v, [tok * E + e])
            sel = v > 0.0
            seli = sel.astype(jnp.int32)
            incl = plsc.cumsum(seli)
            dest = cur[e] + (incl - seli)
            plsc.store_scatter(pos_v, [kcount, tok_local], dest, mask=sel)
            plsc.store_scatter(wflat_v, [tok_local * TOPK + kcount], v, mask=sel)
            kcount = kcount + seli
            cur[e] = cur[e] + jnp.sum(seli)

    for k in range(TOPK):
        pltpu.sync_copy(pos_v.at[k], posT_hbm.at[k, pl.ds(base, TPW)])
    pltpu.sync_copy(wflat_v, wflat_hbm.at[pl.ds(base * TOPK, TPW * TOPK)])

    pltpu.sync_copy(h_hbm.at[pl.ds(base, TPW)], h_v)
    copies = [pltpu.async_copy(h_v, hs_hbm.at[pos_v.at[k]], sem)
              for k in range(TOPK)]
    for cp in copies:
        cp.wait()


def _sc_combine_body(eo_hbm, posT_hbm, g_hbm, idx_v, buf_v, sem):
    wid = lax.axis_index("s") * NC + lax.axis_index("c")
    base = wid * TPW
    for k in range(TOPK):
        pltpu.sync_copy(posT_hbm.at[k, pl.ds(base, TPW)], idx_v.at[k])
    for k in range(TOPK):
        pltpu.async_copy(eo_hbm.at[idx_v.at[k]], buf_v, sem).wait()
        pltpu.sync_copy(buf_v, g_hbm.at[k, pl.ds(base, TPW)])


def _group_body(texp_ref, hs_ref, We1_ref, We3_ref, We2_ref, eo_ref):
    hs = hs_ref[...]
    h1 = jnp.dot(hs, We1_ref[0], preferred_element_type=jnp.float32)
    h3 = jnp.dot(hs, We3_ref[0], preferred_element_type=jnp.float32)
    eo_ref[...] = jnp.dot(_silu(h1) * h3, We2_ref[0],
                          preferred_element_type=jnp.float32)


def _post_body(g_ref, w_ref, S_ref, W1_ref, b1_ref, Wsg1_ref, bsg1_ref,
               W2_ref, b2_ref, Wsg2_ref, bsg2_ref, out_ref):
    y = S_ref[...]
    for k in range(TOPK):
        wk = w_ref[:, k:k + 1]
        y = y + jnp.where(wk > 0.0, wk * g_ref[k], 0.0)
    t1 = jnp.dot(y, W1_ref[...], preferred_element_type=jnp.float32) + b1_ref[...]
    z1 = jnp.dot(t1, Wsg1_ref[...], preferred_element_type=jnp.float32) + bsg1_ref[...]
    y1 = z1[:, :D] * _silu(z1[:, D:])
    t2 = jnp.dot(y1, W2_ref[...], preferred_element_type=jnp.float32) + b2_ref[...]
    z2 = jnp.dot(t2, Wsg2_ref[...], preferred_element_type=jnp.float32) + bsg2_ref[...]
    out_ref[...] = z2[:, :D] * _silu(z2[:, D:])


def kernel(x, W0, b0, Wsg0, bsg0, Wg, We1, We3, We2, Ws1, Ws3, Ws2,
           W1, b1, Wsg1, bsg1, W2, b2, Wsg2, bsg2):
    n_tiles = N // TILE
    full = lambda shape: pl.BlockSpec(shape, lambda t: (0,) * len(shape))
    row_tile = pl.BlockSpec((TILE, D), lambda t: (t, 0))

    h, comb = pl.pallas_call(
        _pre_body,
        grid=(n_tiles,),
        in_specs=[row_tile,
                  full((D, D)), full((D,)), full((D, 2 * D)), full((2 * D,)),
                  full((D, E))],
        out_specs=[row_tile, pl.BlockSpec((TILE, E), lambda t: (t, 0))],
        out_shape=[jax.ShapeDtypeStruct((N, D), jnp.float32),
                   jax.ShapeDtypeStruct((N, E), jnp.float32)],
        compiler_params=pltpu.CompilerParams(
            dimension_semantics=("arbitrary",)),
    )(x, W0, b0, Wsg0, bsg0, Wg)

    S = pl.pallas_call(
        _shared_body,
        grid=(n_tiles,),
        in_specs=[row_tile,
                  full((D, SHID)), full((D, SHID)), full((SHID, D))],
        out_specs=row_tile,
        out_shape=jax.ShapeDtypeStruct((N, D), jnp.float32),
        compiler_params=pltpu.CompilerParams(
            dimension_semantics=("arbitrary",)),
    )(h, Ws1, Ws3, Ws2)

    sc_dispatch = functools.partial(
        pl.kernel,
        out_type=[jax.ShapeDtypeStruct((NPAD, D), jnp.float32),
                  jax.ShapeDtypeStruct((TOPK, N), jnp.int32),
                  jax.ShapeDtypeStruct((N * TOPK,), jnp.float32),
                  jax.ShapeDtypeStruct((NT_G_PAD,), jnp.int32)],
        mesh=plsc.VectorSubcoreMesh(core_axis_name="c", subcore_axis_name="s"),
        scratch_types=[pltpu.VMEM((N * E,), jnp.float32),
                       pltpu.VMEM((TOPK, TPW), jnp.int32),
                       pltpu.VMEM((TPW * TOPK,), jnp.float32),
                       pltpu.VMEM((NT_G_PAD,), jnp.int32),
                       pltpu.VMEM((TPW, D), jnp.float32),
                       pltpu.SemaphoreType.DMA],
        compiler_params=pltpu.CompilerParams(needs_layout_passes=False),
    )(_sc_dispatch_body)
    hs, posT, wflat, texp = sc_dispatch(comb.reshape(N * E), h)

    eo = pl.pallas_call(
        _group_body,
        grid_spec=pltpu.PrefetchScalarGridSpec(
            num_scalar_prefetch=1,
            grid=(NT_G,),
            in_specs=[
                pl.BlockSpec((GT, D), lambda t, texp_r: (t, 0)),
                pl.BlockSpec((1, D, HID), lambda t, texp_r: (texp_r[t], 0, 0)),
                pl.BlockSpec((1, D, HID), lambda t, texp_r: (texp_r[t], 0, 0)),
                pl.BlockSpec((1, HID, D), lambda t, texp_r: (texp_r[t], 0, 0)),
            ],
            out_specs=pl.BlockSpec((GT, D), lambda t, texp_r: (t, 0)),
        ),
        out_shape=jax.ShapeDtypeStruct((NPAD, D), jnp.float32),
        compiler_params=pltpu.CompilerParams(
            dimension_semantics=("arbitrary",)),
    )(texp, hs, We1, We3, We2)

    sc_combine = functools.partial(
        pl.kernel,
        out_type=jax.ShapeDtypeStruct((TOPK, N, D), jnp.float32),
        mesh=plsc.VectorSubcoreMesh(core_axis_name="c", subcore_axis_name="s"),
        scratch_types=[pltpu.VMEM((TOPK, TPW), jnp.int32),
                       pltpu.VMEM((TPW, D), jnp.float32),
                       pltpu.SemaphoreType.DMA],
        compiler_params=pltpu.CompilerParams(needs_layout_passes=False),
    )(_sc_combine_body)
    g = sc_combine(eo, posT)

    out = pl.pallas_call(
        _post_body,
        grid=(n_tiles,),
        in_specs=[pl.BlockSpec((TOPK, TILE, D), lambda t: (0, t, 0)),
                  pl.BlockSpec((TILE, TOPK), lambda t: (t, 0)),
                  row_tile,
                  full((D, D)), full((D,)), full((D, 2 * D)), full((2 * D,)),
                  full((D, D)), full((D,)), full((D, 2 * D)), full((2 * D,))],
        out_specs=row_tile,
        out_shape=jax.ShapeDtypeStruct((N, D), jnp.float32),
        compiler_params=pltpu.CompilerParams(
            dimension_semantics=("arbitrary",)),
    )(g, wflat.reshape(N, TOPK), S, W1, b1, Wsg1, bsg1, W2, b2, Wsg2, bsg2)

    return out
